# baseline (speedup 1.0000x reference)
import functools

import jax
import jax.numpy as jnp
from jax import lax
from jax.experimental import pallas as pl
from jax.experimental.pallas import tpu as pltpu
from jax.experimental.pallas import tpu_sc as plsc

_NC = 2
_NS = 16
_NW = _NC * _NS

_C = 4


def _embed_gather(x, table):
    bsz, h = x.shape
    d = table.shape[1]
    rows_per_w = bsz // _NW
    steps = rows_per_w // _C
    g0 = (h // 2 + 7) & ~7
    g0 = min(g0, 128)
    g1 = h - g0

    mesh = plsc.VectorSubcoreMesh(
        core_axis_name="c", subcore_axis_name="s",
        num_cores=_NC, num_subcores=_NS)

    @functools.partial(
        pl.kernel,
        out_type=jax.ShapeDtypeStruct((bsz, h, d), jnp.float32),
        mesh=mesh,
        scratch_types=[
            pltpu.VMEM((rows_per_w, h), jnp.int32),
            pltpu.VMEM((_C, h, d), jnp.float32),
            pltpu.VMEM((_C, h, d), jnp.float32),
            pltpu.SemaphoreType.DMA,
            pltpu.SemaphoreType.DMA,
            pltpu.SemaphoreType.DMA,
            pltpu.SemaphoreType.DMA,
        ],
        compiler_params=pltpu.CompilerParams(use_tc_tiling_on_sc=False),
    )
    def body(x_hbm, tab_hbm, out_hbm, idx_all, rows0, rows1,
             gsem0, gsem1, osem0, osem1):
        rows = (rows0, rows1)
        gsem = (gsem0, gsem1)
        osem = (osem0, osem1)
        wid = lax.axis_index("s") * _NC + lax.axis_index("c")
        rbase = wid * rows_per_w

        pltpu.sync_copy(x_hbm.at[pl.ds(rbase, rows_per_w)], idx_all)

        def fire(s, b):
            for c in range(_C):
                r = s * _C + c
                pltpu.async_copy(
                    tab_hbm.at[idx_all.at[r, pl.ds(0, g0)]],
                    rows[b].at[c, pl.ds(0, g0)], gsem[b])
                pltpu.async_copy(
                    tab_hbm.at[idx_all.at[r, pl.ds(g0, g1)]],
                    rows[b].at[c, pl.ds(g0, g1)], gsem[b])

        def drain_g(b):
            pltpu.make_async_copy(
                out_hbm.at[pl.ds(0, _C)], rows[b], gsem[b]).wait()

        def start_out(s, b):
            pltpu.async_copy(
                rows[b], out_hbm.at[pl.ds(rbase + s * _C, _C)], osem[b])

        def drain_o(b):
            pltpu.make_async_copy(
                rows[b], out_hbm.at[pl.ds(0, _C)], osem[b]).wait()

        fire(0, 0)

        @pl.loop(0, steps, step=2)
        def _pair(i):
            @pl.when(i > 0)
            def _():
                drain_o(1)
            fire(i + 1, 1)
            drain_g(0)
            start_out(i, 0)

            @pl.when(i + 2 < steps)
            def _():
                drain_o(0)
                fire(i + 2, 0)
            drain_g(1)
            start_out(i + 1, 1)

        drain_o(0)
        drain_o(1)

    return body(x, table)


def kernel(x, embed_weights):
    return _embed_gather(x.astype(jnp.int32), embed_weights)
